# baseline (device time: 40932 ns/iter reference)
import os

import jax
import jax.numpy as jnp
from jax import lax
from jax.experimental import pallas as pl
from jax.experimental.pallas import tpu as pltpu

N_DEV = 16
N_BAND = 8

_VARIANT = os.environ.get("SCBAND_VARIANT", "")
_CDTYPE = (jnp.bfloat16 if os.environ.get("SCBAND_MM_BF16") == "1"
           else jnp.float8_e5m2)


def kernel(x, w_mat, scale_x, scale_w):
    m_per, k = x.shape
    n_tot = w_mat.shape[1]
    n_per = n_tot // N_DEV
    k_band = k // N_BAND
    scale = (scale_x[0] * scale_w[0]).reshape(1, 1).astype(jnp.float32)

    def body(x_ref, w_hbm, scale_ref, out_ref, acc_ref, comm_ref, recv_ref,
             xb_ref, wbuf_ref, load_sems, send_sems, recv_sems):
        me = lax.axis_index("i")
        s = scale_ref[0, 0]

        if _VARIANT != "nocomm":
            barrier_sem = pltpu.get_barrier_semaphore()
            for hop in range(1, N_DEV):
                pl.semaphore_signal(
                    barrier_sem, inc=1,
                    device_id=((me + hop) % N_DEV,),
                    device_id_type=pl.DeviceIdType.MESH,
                )
            pl.semaphore_wait(barrier_sem, N_DEV - 1)

        def w_load(b, slot):
            return pltpu.make_async_copy(
                w_hbm.at[pl.ds(b * k_band, k_band), :],
                wbuf_ref.at[slot],
                load_sems.at[slot],
            )

        w_load(0, 0).start()

        xb_ref[...] = x_ref[...].astype(_CDTYPE)

        for b in range(N_BAND):
            slot = b % 2
            w_load(b, slot).wait()
            if b + 1 < N_BAND:
                w_load(b + 1, slot ^ 1).start()
            partial = jnp.dot(
                xb_ref[:, pl.ds(b * k_band, k_band)],
                wbuf_ref[slot].astype(_CDTYPE),
                preferred_element_type=jnp.float32,
            )
            if b == 0:
                acc_ref[...] = partial
            else:
                acc_ref[...] += partial

        y = acc_ref[...] * s
        acc_ref[...] = y / (1.0 + jnp.exp(-jnp.clip(y, -60.0, 60.0)))

        rdmas = []
        if _VARIANT != "nocomm":
            for dl in range(1, N_DEV):
                d = (me + dl) % N_DEV
                comm_ref[d, :, :] = (
                    acc_ref[:, pl.ds(d * n_per, n_per)].astype(jnp.bfloat16))
                rdma = pltpu.make_async_remote_copy(
                    src_ref=comm_ref.at[d],
                    dst_ref=recv_ref.at[me],
                    send_sem=send_sems.at[d],
                    recv_sem=recv_sems.at[me],
                    device_id=(d,),
                    device_id_type=pl.DeviceIdType.MESH,
                )
                rdma.start()
                rdmas.append(rdma)
        out_ref[pl.ds(me * m_per, m_per), :] = (
            acc_ref[:, pl.ds(me * n_per, n_per)])

        if _VARIANT != "nocomm":
            for sl in range(1, N_DEV):
                src = (me - sl) % N_DEV
                recv = pltpu.make_async_remote_copy(
                    src_ref=comm_ref.at[src],
                    dst_ref=recv_ref.at[src],
                    send_sem=send_sems.at[src],
                    recv_sem=recv_sems.at[src],
                    device_id=(src,),
                    device_id_type=pl.DeviceIdType.MESH,
                )
                recv.wait_recv()
                out_ref[pl.ds(src * m_per, m_per), :] = (
                    recv_ref[src, :, :].astype(jnp.float32))
            for rdma in rdmas:
                rdma.wait_send()
        else:
            for sl in range(1, N_DEV):
                src = (me - sl) % N_DEV
                out_ref[pl.ds(src * m_per, m_per), :] = (
                    recv_ref[src, :, :].astype(jnp.float32))

    return pl.pallas_call(
        body,
        out_shape=jax.ShapeDtypeStruct((N_DEV * m_per, n_per), jnp.float32),
        in_specs=[
            pl.BlockSpec(memory_space=pltpu.VMEM),
            pl.BlockSpec(memory_space=pltpu.MemorySpace.HBM),
            pl.BlockSpec(memory_space=pltpu.SMEM),
        ],
        out_specs=pl.BlockSpec(memory_space=pltpu.VMEM),
        scratch_shapes=[
            pltpu.VMEM((m_per, n_tot), jnp.float32),
            pltpu.VMEM((N_DEV, m_per, n_per), jnp.bfloat16),
            pltpu.VMEM((N_DEV, m_per, n_per), jnp.bfloat16),
            pltpu.VMEM((m_per, k), _CDTYPE),
            pltpu.VMEM((2, k // N_BAND, n_tot), jnp.float32),
            pltpu.SemaphoreType.DMA((2,)),
            pltpu.SemaphoreType.DMA((N_DEV,)),
            pltpu.SemaphoreType.DMA((N_DEV,)),
        ],
        compiler_params=pltpu.CompilerParams(
            collective_id=None if _VARIANT == "nocomm" else 0,
            vmem_limit_bytes=100 * 1024 * 1024,
        ),
    )(x, w_mat, scale)


# device time: 30754 ns/iter; 1.3309x vs baseline; 1.3309x over previous
import os

import jax
import jax.numpy as jnp
from jax import lax
from jax.experimental import pallas as pl
from jax.experimental.pallas import tpu as pltpu

N_DEV = 16
N_CHUNK = int(os.environ.get("SCBAND_NCHUNK", "4"))
D_PER = N_DEV // N_CHUNK

_VARIANT = os.environ.get("SCBAND_VARIANT", "")
_CDTYPE = (jnp.bfloat16 if os.environ.get("SCBAND_MM_BF16") == "1"
           else jnp.float8_e5m2)
_N_SLOTS = int(os.environ.get("SCBAND_NSLOTS", "2"))


def kernel(x, w_mat, scale_x, scale_w):
    m_per, k = x.shape
    n_tot = w_mat.shape[1]
    n_per = n_tot // N_DEV
    n_chunk = n_tot // N_CHUNK
    scale = (scale_x[0] * scale_w[0]).reshape(1, 1).astype(jnp.float32)

    def body(x_ref, w_hbm, scale_ref, out_ref, comm_ref, recv_ref, xb_ref,
             wbuf_ref, load_sems, send_sems, recv_sems):
        me = lax.axis_index("i")
        s = scale_ref[0, 0]

        if _VARIANT not in ("nocomm", "dmaonly"):
            barrier_sem = pltpu.get_barrier_semaphore()
            for hop in range(1, N_DEV):
                pl.semaphore_signal(
                    barrier_sem, inc=1,
                    device_id=((me + hop) % N_DEV,),
                    device_id_type=pl.DeviceIdType.MESH,
                )
            pl.semaphore_wait(barrier_sem, N_DEV - 1)

        def chunk_idx(t):
            return (me // D_PER + 1 + t) % N_CHUNK

        def w_load(t, slot):
            return pltpu.make_async_copy(
                w_hbm.at[:, pl.ds(chunk_idx(t) * n_chunk, n_chunk)],
                wbuf_ref.at[slot],
                load_sems.at[slot],
            )

        if _VARIANT != "nocompute":
            w_load(0, 0).start()

        xb_ref[...] = x_ref[...].astype(_CDTYPE)

        if _VARIANT == "nocompute":
            comm_ref[...] = jnp.zeros_like(comm_ref)

        if _VARIANT == "dmaonly":
            if _N_SLOTS >= N_CHUNK:
                for t in range(1, N_CHUNK):
                    w_load(t, t).start()
                for t in range(N_CHUNK):
                    w_load(t, t if _N_SLOTS >= N_CHUNK else t % 2).wait()
            else:
                for t in range(N_CHUNK):
                    slot = t % 2
                    w_load(t, slot).wait()
                    if t + 1 < N_CHUNK:
                        w_load(t + 1, slot ^ 1).start()
            out_ref[...] = jnp.zeros_like(out_ref)
            return

        rdmas = []
        for t in range(N_CHUNK):
            slot = t % 2
            c = chunk_idx(t)
            if _VARIANT != "nocompute":
                w_load(t, slot).wait()
                if t + 1 < N_CHUNK:
                    w_load(t + 1, slot ^ 1).start()
                acc = jnp.dot(xb_ref[...], wbuf_ref[slot].astype(_CDTYPE),
                              preferred_element_type=jnp.float32)
                y = acc * s
                y = y / (1.0 + jnp.exp(-jnp.clip(y, -60.0, 60.0)))
            for dl in range(D_PER):
                d = c * D_PER + dl
                if _VARIANT != "nocompute":
                    blk = y[:, dl * n_per:(dl + 1) * n_per]

                    @pl.when(d == me)
                    def _(blk=blk):
                        out_ref[pl.ds(me * m_per, m_per), :] = blk

                    @pl.when(d != me)
                    def _(blk=blk, d=d):
                        comm_ref[d, :, :] = blk.astype(jnp.bfloat16)

                if _VARIANT != "nocomm":
                    @pl.when(d != me)
                    def _(d=d):
                        rdma = pltpu.make_async_remote_copy(
                            src_ref=comm_ref.at[d],
                            dst_ref=recv_ref.at[me],
                            send_sem=send_sems.at[d],
                            recv_sem=recv_sems.at[me],
                            device_id=(d,),
                            device_id_type=pl.DeviceIdType.MESH,
                        )
                        rdma.start()

        for j in range(1, N_CHUNK + 1):
            for i in range(D_PER):
                src = ((me // D_PER + j) % N_CHUNK) * D_PER + i
                if _VARIANT != "nocomm":
                    @pl.when(src != me)
                    def _(src=src):
                        recv = pltpu.make_async_remote_copy(
                            src_ref=comm_ref.at[src],
                            dst_ref=recv_ref.at[src],
                            send_sem=send_sems.at[src],
                            recv_sem=recv_sems.at[src],
                            device_id=(src,),
                            device_id_type=pl.DeviceIdType.MESH,
                        )
                        recv.wait_recv()
                        out_ref[pl.ds(src * m_per, m_per), :] = (
                            recv_ref[src, :, :].astype(jnp.float32))
                else:
                    @pl.when(src != me)
                    def _(src=src):
                        out_ref[pl.ds(src * m_per, m_per), :] = (
                            recv_ref[src, :, :].astype(jnp.float32))

        if _VARIANT != "nocomm":
            for d in range(N_DEV):
                @pl.when(d != me)
                def _(d=d):
                    pltpu.make_async_remote_copy(
                        src_ref=comm_ref.at[d],
                        dst_ref=recv_ref.at[d],
                        send_sem=send_sems.at[d],
                        recv_sem=recv_sems.at[d],
                        device_id=(d,),
                        device_id_type=pl.DeviceIdType.MESH,
                    ).wait_send()

    return pl.pallas_call(
        body,
        out_shape=jax.ShapeDtypeStruct((N_DEV * m_per, n_per), jnp.float32),
        in_specs=[
            pl.BlockSpec(memory_space=pltpu.VMEM),
            pl.BlockSpec(memory_space=pltpu.MemorySpace.HBM),
            pl.BlockSpec(memory_space=pltpu.SMEM),
        ],
        out_specs=pl.BlockSpec(memory_space=pltpu.VMEM),
        scratch_shapes=[
            pltpu.VMEM((N_DEV, m_per, n_per), jnp.bfloat16),
            pltpu.VMEM((N_DEV, m_per, n_per), jnp.bfloat16),
            pltpu.VMEM((m_per, k), _CDTYPE),
            pltpu.VMEM((_N_SLOTS, k, n_tot // N_CHUNK), jnp.float32),
            pltpu.SemaphoreType.DMA((_N_SLOTS,)),
            pltpu.SemaphoreType.DMA((N_DEV,)),
            pltpu.SemaphoreType.DMA((N_DEV,)),
        ],
        compiler_params=pltpu.CompilerParams(
            collective_id=(None if _VARIANT in ("nocomm", "dmaonly") else 0),
            vmem_limit_bytes=100 * 1024 * 1024,
        ),
    )(x, w_mat, scale)
